# baseline (speedup 1.0000x reference)
import functools

import jax
import jax.numpy as jnp
from jax import lax
from jax.experimental import pallas as pl
from jax.experimental.pallas import tpu as pltpu
from jax.experimental.pallas import tpu_sc as plsc

NUM_EMBEDDINGS = 100000
D = 768
CONTEXT_LENGTH = 2048
BATCH = 4
B_TOTAL = BATCH * CONTEXT_LENGTH

NC, NS = 2, 16
NW = NC * NS
B_PER_W = B_TOTAL // NW
CHUNK = 32
N_CHUNKS = B_PER_W // CHUNK
LANES = 16
VECS_PER_ROW = D // LANES


def _embed_body(x_hbm, tok_hbm, pos_hbm, out_hbm, idx_v, rows_v, pos_v,
                sem_g, sem_p):
    wid = lax.axis_index("s") * NC + lax.axis_index("c")
    base = wid * B_PER_W
    pos_base = lax.rem(base, CONTEXT_LENGTH)

    pltpu.sync_copy(x_hbm.at[wid], idx_v)

    for c in range(N_CHUNKS):
        row0 = base + c * CHUNK
        g = pltpu.async_copy(tok_hbm.at[idx_v.at[c]], rows_v, sem_g)
        p = pltpu.async_copy(
            pos_hbm.at[pl.ds(pos_base + c * CHUNK, CHUNK)], pos_v, sem_p)
        g.wait()
        p.wait()

        def row_body(r, carry):
            for v in range(VECS_PER_ROW):
                sl = pl.ds(v * LANES, LANES)
                plsc.addupdate(rows_v.at[r, sl], pos_v[r, sl])
            return carry

        lax.fori_loop(0, CHUNK, row_body, 0)
        pltpu.sync_copy(rows_v, out_hbm.at[pl.ds(row0, CHUNK)])


@functools.partial(jax.jit, static_argnames=())
def _embed(x_grouped, tok_emb_weight, pos_emb_weight):
    mesh = plsc.VectorSubcoreMesh(
        core_axis_name="c", subcore_axis_name="s", num_cores=NC,
        num_subcores=NS)
    return pl.kernel(
        _embed_body,
        out_type=jax.ShapeDtypeStruct((B_TOTAL, D), jnp.float32),
        mesh=mesh,
        scratch_types=[
            pltpu.VMEM((N_CHUNKS, CHUNK), jnp.int32),
            pltpu.VMEM((CHUNK, D), jnp.float32),
            pltpu.VMEM((CHUNK, D), jnp.float32),
            pltpu.SemaphoreType.DMA,
            pltpu.SemaphoreType.DMA,
        ],
    )(x_grouped, tok_emb_weight, pos_emb_weight)


def kernel(x, tok_emb_weight, pos_emb_weight):
    batch, cxt = x.shape
    x_grouped = x.astype(jnp.int32).reshape(NW, N_CHUNKS, CHUNK)
    out = _embed(x_grouped, tok_emb_weight, pos_emb_weight)
    return out.reshape(batch, cxt, D)


# speedup vs baseline: 1.1206x; 1.1206x over previous
import functools

import jax
import jax.numpy as jnp
from jax import lax
from jax.experimental import pallas as pl
from jax.experimental.pallas import tpu as pltpu
from jax.experimental.pallas import tpu_sc as plsc

NUM_EMBEDDINGS = 100000
D = 768
CONTEXT_LENGTH = 2048
BATCH = 4
B_TOTAL = BATCH * CONTEXT_LENGTH

NC, NS = 2, 16
NW = NC * NS
POS_PER_W = CONTEXT_LENGTH // NW
CHUNK = 32
HALVES = POS_PER_W // CHUNK
N_CHUNKS = BATCH * HALVES
LANES = 16
VECS_PER_ROW = D // LANES


def _embed_body(x_hbm, tok_hbm, pos_hbm, out_hbm, idx_v, rows_v, pos_v,
                sem_g0, sem_g1, sem_o0, sem_o1):
    wid = lax.axis_index("s") * NC + lax.axis_index("c")
    p0 = wid * POS_PER_W

    sem_g = (sem_g0, sem_g1)
    sem_o = (sem_o0, sem_o1)

    pltpu.sync_copy(x_hbm.at[wid], idx_v)
    pltpu.sync_copy(pos_hbm.at[pl.ds(p0, POS_PER_W)], pos_v)

    def gather(c):
        b, h = divmod(c, HALVES)
        return pltpu.async_copy(
            tok_hbm.at[idx_v.at[b, h]], rows_v.at[c % 2], sem_g[c % 2])

    copies = {0: gather(0)}
    out_copies = {}
    for c in range(N_CHUNKS):
        b, h = divmod(c, HALVES)
        if c + 1 < N_CHUNKS:
            if c - 1 >= 0:
                out_copies[c - 1].wait()
            copies[c + 1] = gather(c + 1)
        copies[c].wait()

        buf = rows_v.at[c % 2]
        ph = h * CHUNK

        def row_body(r, carry):
            for v in range(VECS_PER_ROW):
                sl = pl.ds(v * LANES, LANES)
                plsc.addupdate(buf.at[r, sl], pos_v[ph + r, sl])
            return carry

        lax.fori_loop(0, CHUNK, row_body, 0)

        row0 = b * CONTEXT_LENGTH + p0 + ph
        out_copies[c] = pltpu.async_copy(
            buf, out_hbm.at[pl.ds(row0, CHUNK)], sem_o[c % 2])
    out_copies[N_CHUNKS - 2].wait()
    out_copies[N_CHUNKS - 1].wait()


@jax.jit
def _embed(x_grouped, tok_emb_weight, pos_emb_weight):
    mesh = plsc.VectorSubcoreMesh(
        core_axis_name="c", subcore_axis_name="s", num_cores=NC,
        num_subcores=NS)
    return pl.kernel(
        _embed_body,
        out_type=jax.ShapeDtypeStruct((B_TOTAL, D), jnp.float32),
        mesh=mesh,
        scratch_types=[
            pltpu.VMEM((NW // NW * BATCH, HALVES, CHUNK), jnp.int32),
            pltpu.VMEM((2, CHUNK, D), jnp.float32),
            pltpu.VMEM((POS_PER_W, D), jnp.float32),
            pltpu.SemaphoreType.DMA,
            pltpu.SemaphoreType.DMA,
            pltpu.SemaphoreType.DMA,
            pltpu.SemaphoreType.DMA,
        ],
    )(x_grouped, tok_emb_weight, pos_emb_weight)


def kernel(x, tok_emb_weight, pos_emb_weight):
    batch, cxt = x.shape
    x_grouped = (x.astype(jnp.int32)
                 .reshape(batch, NW, HALVES, CHUNK)
                 .transpose(1, 0, 2, 3))
    out = _embed(x_grouped, tok_emb_weight, pos_emb_weight)
    return out.reshape(batch, cxt, D)


# speedup vs baseline: 1.1275x; 1.0061x over previous
import functools

import jax
import jax.numpy as jnp
from jax import lax
from jax.experimental import pallas as pl
from jax.experimental.pallas import tpu as pltpu
from jax.experimental.pallas import tpu_sc as plsc

NUM_EMBEDDINGS = 100000
D = 768
CONTEXT_LENGTH = 2048
BATCH = 4
B_TOTAL = BATCH * CONTEXT_LENGTH

NC, NS = 2, 16
NW = NC * NS
POS_PER_W = CONTEXT_LENGTH // NW
CHUNK = 32
HALVES = POS_PER_W // CHUNK
N_CHUNKS = BATCH * HALVES
LANES = 16
VECS_PER_ROW = D // LANES


NBUF = 3


def _embed_body(x_hbm, tok_hbm, pos_hbm, out_hbm, idx_v, rows_v, pos_v,
                sem_g0, sem_g1, sem_g2, sem_o0, sem_o1, sem_o2):
    wid = lax.axis_index("s") * NC + lax.axis_index("c")
    p0 = wid * POS_PER_W

    sem_g = (sem_g0, sem_g1, sem_g2)
    sem_o = (sem_o0, sem_o1, sem_o2)

    pltpu.sync_copy(x_hbm.at[wid], idx_v)
    pltpu.sync_copy(pos_hbm.at[pl.ds(p0, POS_PER_W)], pos_v)

    def gather(c):
        b, h = divmod(c, HALVES)
        return pltpu.async_copy(
            tok_hbm.at[idx_v.at[b, h]], rows_v.at[c % NBUF], sem_g[c % NBUF])

    copies = {0: gather(0), 1: gather(1)}
    out_copies = {}
    for c in range(N_CHUNKS):
        b, h = divmod(c, HALVES)
        if c + 2 < N_CHUNKS:
            if c - 1 >= 0:
                out_copies[c - 1].wait()
            copies[c + 2] = gather(c + 2)
        copies[c].wait()

        buf = rows_v.at[c % NBUF]
        ph = h * CHUNK

        def row_body(r, carry):
            for v in range(VECS_PER_ROW):
                sl = pl.ds(v * LANES, LANES)
                plsc.addupdate(buf.at[r, sl], pos_v[ph + r, sl])
            return carry

        lax.fori_loop(0, CHUNK, row_body, 0)

        row0 = b * CONTEXT_LENGTH + p0 + ph
        out_copies[c] = pltpu.async_copy(
            buf, out_hbm.at[pl.ds(row0, CHUNK)], sem_o[c % NBUF])
    for c in range(max(0, N_CHUNKS - 3), N_CHUNKS):
        out_copies[c].wait()


@jax.jit
def _embed(x_grouped, tok_emb_weight, pos_emb_weight):
    mesh = plsc.VectorSubcoreMesh(
        core_axis_name="c", subcore_axis_name="s", num_cores=NC,
        num_subcores=NS)
    return pl.kernel(
        _embed_body,
        out_type=jax.ShapeDtypeStruct((B_TOTAL, D), jnp.float32),
        mesh=mesh,
        scratch_types=[
            pltpu.VMEM((BATCH, HALVES, CHUNK), jnp.int32),
            pltpu.VMEM((NBUF, CHUNK, D), jnp.float32),
            pltpu.VMEM((POS_PER_W, D), jnp.float32),
            pltpu.SemaphoreType.DMA,
            pltpu.SemaphoreType.DMA,
            pltpu.SemaphoreType.DMA,
            pltpu.SemaphoreType.DMA,
            pltpu.SemaphoreType.DMA,
            pltpu.SemaphoreType.DMA,
        ],
    )(x_grouped, tok_emb_weight, pos_emb_weight)


def kernel(x, tok_emb_weight, pos_emb_weight):
    batch, cxt = x.shape
    x_grouped = (x.astype(jnp.int32)
                 .reshape(batch, NW, HALVES, CHUNK)
                 .transpose(1, 0, 2, 3))
    out = _embed(x_grouped, tok_emb_weight, pos_emb_weight)
    return out.reshape(batch, cxt, D)
